# baseline (speedup 1.0000x reference)
import functools

import jax
import jax.numpy as jnp
from jax import lax
from jax.experimental import pallas as pl
from jax.experimental.pallas import tpu as pltpu
from jax.experimental.pallas import tpu_sc as plsc

N_ENTITIES = 1000000
N_PREDICATES = 1000
D = 64
B = 16384

NC = 2
NS = 16
L = 16
NW = NC * NS
BPW = B // NW
CH = 128
NCHUNK = BPW // CH
GPC = CH // L


def _sc_body(head_hbm, rel_hbm, tail_hbm, ent_hbm, relt_hbm, out_hbm,
             hidx, ridx, tidx, hb, rb, tb, outv, sem):
    wid = lax.axis_index("s") * NC + lax.axis_index("c")
    base = wid * BPW

    pltpu.sync_copy(head_hbm.at[pl.ds(base, BPW)], hidx)
    pltpu.sync_copy(rel_hbm.at[pl.ds(base, BPW)], ridx)
    pltpu.sync_copy(tail_hbm.at[pl.ds(base, BPW)], tidx)

    lanes = lax.iota(jnp.int32, L)

    def chunk(c, carry):
        c0 = c * CH
        copies = []
        for g16 in range(GPC):
            gsl = pl.ds(c0 + g16 * L, L)
            hv = hidx[gsl]
            rv = ridx[gsl]
            tv = tidx[gsl]
            for k16 in range(L):
                k = g16 * L + k16
                copies.append(pltpu.async_copy(ent_hbm.at[0, hv[k16]], hb.at[k], sem))
                copies.append(pltpu.async_copy(relt_hbm.at[rv[k16]], rb.at[k], sem))
                copies.append(pltpu.async_copy(ent_hbm.at[0, tv[k16]], tb.at[k], sem))
        for cp in copies:
            cp.wait()

        def group(g, gcarry):
            row0 = g * L
            acc = jnp.zeros((L,), jnp.float32)
            for i in range(L):
                part = jnp.zeros((L,), jnp.float32)
                for j in range(D // L):
                    sl = pl.ds(j * L, L)
                    d = hb[row0 + i, sl] * rb[row0 + i, sl] - tb[row0 + i, sl]
                    part = part + d * d
                acc = jnp.where(lanes == i, jnp.sum(part), acc)
            bits = lax.bitcast_convert_type(acc, jnp.int32)
            zb = jnp.int32(0x5F3759DF) - lax.shift_right_logical(bits, 1)
            z = lax.bitcast_convert_type(zb, jnp.float32)
            for _ in range(3):
                z = z * (jnp.float32(1.5) - jnp.float32(0.5) * acc * z * z)
            outv[pl.ds(c0 + row0, L)] = -(acc * z)
            return gcarry

        lax.fori_loop(0, GPC, group, 0)
        return carry

    lax.fori_loop(0, NCHUNK, chunk, 0)
    pltpu.sync_copy(outv, out_hbm.at[pl.ds(base, BPW)])


@jax.jit
def _score(head, relation, tail, entity_embeddings, relation_embeddings):
    ent3 = entity_embeddings.reshape(1, N_ENTITIES, D)
    mesh = plsc.VectorSubcoreMesh(core_axis_name="c", subcore_axis_name="s")
    run = functools.partial(
        pl.kernel,
        out_type=jax.ShapeDtypeStruct((B,), jnp.float32),
        mesh=mesh,
        compiler_params=pltpu.CompilerParams(
            needs_layout_passes=False, use_tc_tiling_on_sc=True
        ),
        scratch_types=[
            pltpu.VMEM((BPW,), jnp.int32),
            pltpu.VMEM((BPW,), jnp.int32),
            pltpu.VMEM((BPW,), jnp.int32),
            pltpu.VMEM((CH, D), jnp.float32),
            pltpu.VMEM((CH, D), jnp.float32),
            pltpu.VMEM((CH, D), jnp.float32),
            pltpu.VMEM((BPW,), jnp.float32),
            pltpu.SemaphoreType.DMA,
        ],
    )(_sc_body)
    return run(head, relation, tail, ent3, relation_embeddings)


def kernel(head, relation, tail, entity_embeddings, relation_embeddings):
    return _score(
        head.astype(jnp.int32),
        relation.astype(jnp.int32),
        tail.astype(jnp.int32),
        entity_embeddings,
        relation_embeddings,
    )


# speedup vs baseline: 1.0297x; 1.0297x over previous
import functools

import jax
import jax.numpy as jnp
from jax import lax
from jax.experimental import pallas as pl
from jax.experimental.pallas import tpu as pltpu
from jax.experimental.pallas import tpu_sc as plsc

N_ENTITIES = 1000000
N_PREDICATES = 1000
D = 64
B = 16384

NC = 2
NS = 16
L = 16
NW = NC * NS
BPW = B // NW
CH = 32
NCHUNK = BPW // CH
GPC = CH // L


def _sc_body(head_hbm, rel_hbm, tail_hbm, ent_hbm, relt_hbm, out_hbm,
             hidx, ridx, tidx, hb, rb, tb, outv, sem):
    wid = lax.axis_index("s") * NC + lax.axis_index("c")
    base = wid * BPW

    pltpu.sync_copy(head_hbm.at[pl.ds(base, BPW)], hidx)
    pltpu.sync_copy(rel_hbm.at[pl.ds(base, BPW)], ridx)
    pltpu.sync_copy(tail_hbm.at[pl.ds(base, BPW)], tidx)

    lanes = lax.iota(jnp.int32, L)

    def chunk(c, carry):
        c0 = c * CH
        group_copies = []
        for g16 in range(GPC):
            gsl = pl.ds(c0 + g16 * L, L)
            hv = hidx[gsl]
            rv = ridx[gsl]
            tv = tidx[gsl]
            copies = []
            for k16 in range(L):
                k = g16 * L + k16
                copies.append(pltpu.async_copy(ent_hbm.at[0, hv[k16]], hb.at[k], sem))
                copies.append(pltpu.async_copy(relt_hbm.at[rv[k16]], rb.at[k], sem))
                copies.append(pltpu.async_copy(ent_hbm.at[0, tv[k16]], tb.at[k], sem))
            group_copies.append(copies)

        for g in range(GPC):
            for cp in group_copies[g]:
                cp.wait()
            row0 = g * L
            acc = jnp.zeros((L,), jnp.float32)
            for i in range(L):
                part = jnp.zeros((L,), jnp.float32)
                for j in range(D // L):
                    sl = pl.ds(j * L, L)
                    d = hb[row0 + i, sl] * rb[row0 + i, sl] - tb[row0 + i, sl]
                    part = part + d * d
                acc = jnp.where(lanes == i, jnp.sum(part), acc)
            bits = lax.bitcast_convert_type(acc, jnp.int32)
            zb = jnp.int32(0x5F3759DF) - lax.shift_right_logical(bits, 1)
            z = lax.bitcast_convert_type(zb, jnp.float32)
            for _ in range(3):
                z = z * (jnp.float32(1.5) - jnp.float32(0.5) * acc * z * z)
            outv[pl.ds(c0 + row0, L)] = -(acc * z)
        return carry

    lax.fori_loop(0, NCHUNK, chunk, 0)
    pltpu.sync_copy(outv, out_hbm.at[pl.ds(base, BPW)])


@jax.jit
def _score(head, relation, tail, entity_embeddings, relation_embeddings):
    ent3 = entity_embeddings.reshape(1, N_ENTITIES, D)
    mesh = plsc.VectorSubcoreMesh(core_axis_name="c", subcore_axis_name="s")
    run = functools.partial(
        pl.kernel,
        out_type=jax.ShapeDtypeStruct((B,), jnp.float32),
        mesh=mesh,
        compiler_params=pltpu.CompilerParams(
            needs_layout_passes=False, use_tc_tiling_on_sc=True
        ),
        scratch_types=[
            pltpu.VMEM((BPW,), jnp.int32),
            pltpu.VMEM((BPW,), jnp.int32),
            pltpu.VMEM((BPW,), jnp.int32),
            pltpu.VMEM((CH, D), jnp.float32),
            pltpu.VMEM((CH, D), jnp.float32),
            pltpu.VMEM((CH, D), jnp.float32),
            pltpu.VMEM((BPW,), jnp.float32),
            pltpu.SemaphoreType.DMA,
        ],
    )(_sc_body)
    return run(head, relation, tail, ent3, relation_embeddings)


def kernel(head, relation, tail, entity_embeddings, relation_embeddings):
    return _score(
        head.astype(jnp.int32),
        relation.astype(jnp.int32),
        tail.astype(jnp.int32),
        entity_embeddings,
        relation_embeddings,
    )
